# baseline (speedup 1.0000x reference)
import functools
import jax
import jax.numpy as jnp
from jax.experimental import pallas as pl
from jax.experimental.pallas import tpu as pltpu
from jax.experimental.pallas import tpu_sc as plsc

_R = 160


def kernel(x, mask, w):
    n, d = x.shape
    nblk = n // _R
    m = mask.astype(jnp.float32)

    mesh = plsc.VectorSubcoreMesh(core_axis_name="core", subcore_axis_name="subcore")

    @functools.partial(
        pl.kernel,
        out_type=jax.ShapeDtypeStruct((n, d), jnp.float32),
        mesh=mesh,
        scratch_types=[pltpu.VMEM((d,), jnp.float32)],
    )
    def run(x_hbm, m_hbm, w_hbm, o_hbm, wv):
        pltpu.sync_copy(w_hbm, wv)
        wvecs = [wv[pl.ds(16 * j, 16)] for j in range(8)]

        def body(x_vmem, m_vmem, o_vmem):
            @pl.loop(0, _R // 16)
            def _(g):
                mvec = m_vmem[pl.ds(16 * g, 16)]
                for l in range(16):
                    r = 16 * g + l
                    mv = mvec[l]
                    for j in range(8):
                        sl = pl.ds(16 * j, 16)
                        o_vmem[r, sl] = x_vmem[r, sl] + mv * wvecs[j]

        pltpu.emit_pipeline(
            body,
            grid=(nblk,),
            in_specs=[
                pl.BlockSpec((_R, d), lambda i: (i, 0)),
                pl.BlockSpec((_R,), lambda i: (i,)),
            ],
            out_specs=[pl.BlockSpec((_R, d), lambda i: (i, 0))],
            core_axis_name=("core", "subcore"),
            dimension_semantics=(pltpu.PARALLEL,),
        )(x_hbm, m_hbm, o_hbm)

    return run(x, m, w)


# speedup vs baseline: 3.1218x; 3.1218x over previous
import functools
import jax
import jax.numpy as jnp
from jax import lax
from jax.experimental import pallas as pl
from jax.experimental.pallas import tpu as pltpu
from jax.experimental.pallas import tpu_sc as plsc

_R = 320
_NW = 32
_NS = 3


def kernel(x, mask, w):
    n, d = x.shape
    nblk = n // _R
    nvisit = ((nblk + _NW - 1) // _NW + _NS - 1) // _NS * _NS
    m = mask.astype(jnp.float32)

    mesh = plsc.VectorSubcoreMesh(core_axis_name="core", subcore_axis_name="subcore")

    @functools.partial(
        pl.kernel,
        out_type=jax.ShapeDtypeStruct((n, d), jnp.float32),
        mesh=mesh,
        scratch_types=(
            [pltpu.VMEM((_R, d), jnp.float32) for _ in range(_NS)]
            + [pltpu.VMEM((_R,), jnp.float32) for _ in range(_NS)]
            + [pltpu.VMEM((d,), jnp.float32)]
            + [pltpu.SemaphoreType.DMA for _ in range(3 * _NS)]
        ),
    )
    def run(x_hbm, m_hbm, w_hbm, o_hbm, *scratch):
        bufs = scratch[0:_NS]
        mbufs = scratch[_NS:2 * _NS]
        wv = scratch[2 * _NS]
        xsems = scratch[2 * _NS + 1:2 * _NS + 1 + _NS]
        msems = scratch[2 * _NS + 1 + _NS:2 * _NS + 1 + 2 * _NS]
        osems = scratch[2 * _NS + 1 + 2 * _NS:]

        pltpu.sync_copy(w_hbm, wv)
        wvecs = [wv[pl.ds(16 * j, 16)] for j in range(8)]

        wid = lax.axis_index("subcore") * 2 + lax.axis_index("core")

        def start_in(s, k):
            b = wid + k * _NW
            base = b * _R
            pltpu.async_copy(x_hbm.at[pl.ds(base, _R), :], bufs[s], xsems[s])
            pltpu.async_copy(m_hbm.at[pl.ds(base, _R)], mbufs[s], msems[s])

        def wait_in(s):
            pltpu.make_async_copy(x_hbm.at[pl.ds(0, _R), :], bufs[s], xsems[s]).wait()
            pltpu.make_async_copy(m_hbm.at[pl.ds(0, _R)], mbufs[s], msems[s]).wait()

        def wait_out(s):
            pltpu.make_async_copy(bufs[s], o_hbm.at[pl.ds(0, _R), :], osems[s]).wait()

        @pl.when(wid < nblk)
        def _():
            start_in(0, 0)

        @pl.loop(0, nvisit, step=_NS)
        def _(i):
            for s in range(_NS):
                k = i + s
                b = wid + k * _NW
                knext = k + 1
                bnext = wid + knext * _NW
                snext = (s + 1) % _NS

                @pl.when(bnext < nblk)
                def _(snext=snext, knext=knext):
                    @pl.when(knext >= _NS)
                    def _():
                        wait_out(snext)

                    start_in(snext, knext)

                @pl.when(b < nblk)
                def _(s=s, k=k, b=b):
                    wait_in(s)
                    buf = bufs[s]
                    mb = mbufs[s]

                    @plsc.parallel_loop(0, _R // 16, unroll=2)
                    def _(g):
                        mvec = mb[pl.ds(16 * g, 16)]
                        for l in range(16):
                            r = 16 * g + l
                            mv = mvec[l]
                            for j in range(8):
                                sl = pl.ds(16 * j, 16)
                                buf[r, sl] = buf[r, sl] + mv * wvecs[j]

                    base = b * _R
                    pltpu.async_copy(buf, o_hbm.at[pl.ds(base, _R), :], osems[s])

        for s in range(_NS):
            nb_w = (nblk - wid + _NW - 1) // _NW

            @pl.when(nb_w > s)
            def _(s=s):
                wait_out(s)

    return run(x, m, w)
